# baseline (speedup 1.0000x reference)
import math

import jax
import jax.numpy as jnp
import numpy as np
from jax.experimental import pallas as pl
from jax.experimental.pallas import tpu as pltpu
from jax.sharding import Mesh, PartitionSpec as P

E = 8
D = 768
H = 2 * D
T = 32768
TB = 2048
TEMPERATURE = 1.0

_bf16 = jnp.bfloat16
_f32 = jnp.float32

_devs = jax.devices()
NDEV = 2 if len(_devs) >= 2 else 1
_MESH = Mesh(np.array(_devs[:NDEV]), ("t",))
TL = T // NDEV
NBL = TL // TB
EL = E // NDEV


def _phase_a_kernel(x_ref, wr_ref, logits_ref, m_ref, l_ref, s_ref):
    i = pl.program_id(0)

    @pl.when(i == 0)
    def _init():
        m_ref[...] = jnp.full_like(m_ref[...], -1e30)
        l_ref[...] = jnp.zeros_like(l_ref[...])
        s_ref[...] = jnp.zeros_like(s_ref[...])

    x = x_ref[...]
    logits = jax.lax.dot_general(
        wr_ref[...].astype(_bf16), x.astype(_bf16),
        (((1,), (1,)), ((), ())), preferred_element_type=_f32)
    logits_ref[0] = logits
    m_old = m_ref[...]
    bmax = jnp.max(logits, axis=1, keepdims=True)
    m_new = jnp.maximum(m_old, bmax)
    scale = jnp.exp(m_old - m_new)
    p = jnp.exp(logits - m_new)
    m_ref[...] = m_new
    l_ref[...] = l_ref[...] * scale + jnp.sum(p, axis=1, keepdims=True)
    s_ref[...] = s_ref[...] * scale + jax.lax.dot_general(
        p.astype(_bf16), x.astype(_bf16),
        (((1,), (0,)), ((), ())), preferred_element_type=_f32)


def _phase_b_kernel(slots_ref, w1_ref, b1_ref, g_ref, beta_ref, w2_ref,
                    b2_ref, out_ref):
    s = slots_ref[0]
    h = jax.lax.dot_general(
        s.astype(_bf16), w1_ref[0].astype(_bf16),
        (((1,), (1,)), ((), ())), preferred_element_type=_f32)
    h = h + b1_ref[0]
    mu = jnp.mean(h, axis=1, keepdims=True)
    var = jnp.mean((h - mu) ** 2, axis=1, keepdims=True)
    h = (h - mu) * jax.lax.rsqrt(var + 1e-5) * g_ref[0] + beta_ref[0]
    h = 0.5 * h * (1.0 + jax.lax.erf(h * (1.0 / math.sqrt(2.0))))
    out = jax.lax.dot_general(
        h.astype(_bf16), w2_ref[0].astype(_bf16),
        (((1,), (1,)), ((), ())), preferred_element_type=_f32)
    out_ref[0] = out + b2_ref[0]


def _phase_c_kernel(logits_ref, so_ref, wr_ref, out_ref, loss_ref, usage_s):
    j = pl.program_id(0)
    nb = pl.num_programs(0)

    @pl.when(j == 0)
    def _init():
        usage_s[...] = jnp.zeros_like(usage_s[...])

    logits = logits_ref[0]
    mx = jnp.max(logits, axis=0, keepdims=True)
    p = jnp.exp(logits - mx)
    combine = p / jnp.sum(p, axis=0, keepdims=True)
    out_ref[...] = jax.lax.dot_general(
        combine.astype(_bf16), so_ref[...].astype(_bf16),
        (((0,), (0,)), ((), ())), preferred_element_type=_f32)
    usage_s[...] += jnp.sum(combine, axis=1, keepdims=True)

    @pl.when(j == nb - 1)
    def _loss():
        rm = jnp.mean(wr_ref[...], axis=1, keepdims=True)
        pe = jnp.exp(rm - jnp.max(rm, axis=0, keepdims=True))
        pe = pe / jnp.sum(pe, axis=0, keepdims=True)
        loss_ref[...] = (float(E) / float(T)) * jnp.sum(
            usage_s[...] * pe, keepdims=True)


def _shard_fn(x, wr, w1, b1, g, beta, w2, b2):
    logits, m, l, s = pl.pallas_call(
        _phase_a_kernel,
        grid=(NBL,),
        in_specs=[
            pl.BlockSpec((TB, D), lambda i: (i, 0)),
            pl.BlockSpec((E, D), lambda i: (0, 0)),
        ],
        out_specs=[
            pl.BlockSpec((1, E, TB), lambda i: (i, 0, 0)),
            pl.BlockSpec((E, 1), lambda i: (0, 0)),
            pl.BlockSpec((E, 1), lambda i: (0, 0)),
            pl.BlockSpec((E, D), lambda i: (0, 0)),
        ],
        out_shape=[
            jax.ShapeDtypeStruct((NBL, E, TB), _f32),
            jax.ShapeDtypeStruct((E, 1), _f32),
            jax.ShapeDtypeStruct((E, 1), _f32),
            jax.ShapeDtypeStruct((E, D), _f32),
        ],
    )(x, wr)

    m_g = jax.lax.pmax(m, "t")
    adj = jnp.exp(m - m_g)
    l_g = jax.lax.psum(l * adj, "t")
    s_g = jax.lax.psum(s * adj, "t")
    slots_in = s_g / l_g

    idx = jax.lax.axis_index("t")
    slots_l = jax.lax.dynamic_slice_in_dim(slots_in, idx * EL, EL, 0)

    so_l = pl.pallas_call(
        _phase_b_kernel,
        grid=(EL,),
        in_specs=[
            pl.BlockSpec((1, 1, D), lambda e: (e, 0, 0)),
            pl.BlockSpec((1, H, D), lambda e: (e, 0, 0)),
            pl.BlockSpec((1, 1, H), lambda e: (e, 0, 0)),
            pl.BlockSpec((1, 1, H), lambda e: (e, 0, 0)),
            pl.BlockSpec((1, 1, H), lambda e: (e, 0, 0)),
            pl.BlockSpec((1, D, H), lambda e: (e, 0, 0)),
            pl.BlockSpec((1, 1, D), lambda e: (e, 0, 0)),
        ],
        out_specs=pl.BlockSpec((1, 1, D), lambda e: (e, 0, 0)),
        out_shape=jax.ShapeDtypeStruct((EL, 1, D), _f32),
    )(slots_l.reshape(EL, 1, D), w1, b1, g, beta, w2, b2)

    so = jax.lax.all_gather(so_l.reshape(EL, D), "t", axis=0, tiled=True)

    out, loss_p = pl.pallas_call(
        _phase_c_kernel,
        grid=(NBL,),
        in_specs=[
            pl.BlockSpec((1, E, TB), lambda j: (j, 0, 0)),
            pl.BlockSpec((E, D), lambda j: (0, 0)),
            pl.BlockSpec((E, D), lambda j: (0, 0)),
        ],
        out_specs=[
            pl.BlockSpec((TB, D), lambda j: (j, 0)),
            pl.BlockSpec((1, 1), lambda j: (0, 0)),
        ],
        out_shape=[
            jax.ShapeDtypeStruct((TL, D), _f32),
            jax.ShapeDtypeStruct((1, 1), _f32),
        ],
        scratch_shapes=[pltpu.VMEM((E, 1), _f32)],
    )(logits, so, wr)

    loss = jax.lax.psum(loss_p, "t")
    return out, loss


@jax.jit
def kernel(x, Wr, W1, b1, g, beta, W2, b2):
    fn = jax.shard_map(
        _shard_fn,
        mesh=_MESH,
        in_specs=(P("t", None), P(None, None), P("t", None, None),
                  P("t", None, None), P("t", None, None), P("t", None, None),
                  P("t", None, None), P("t", None, None)),
        out_specs=(P("t", None), P(None, None)),
        check_vma=False,
    )
    output, loss = fn(x, Wr, W1, b1.reshape(E, 1, H), g.reshape(E, 1, H),
                      beta.reshape(E, 1, H), W2, b2.reshape(E, 1, D))
    return (output, loss.reshape(()))


# speedup vs baseline: 7.5106x; 7.5106x over previous
import math

import jax
import jax.numpy as jnp
from jax.experimental import pallas as pl
from jax.experimental.pallas import tpu as pltpu

E = 8
D = 768
H = 2 * D
T = 32768
TB = 1024
NB = T // TB

_bf16 = jnp.bfloat16
_f32 = jnp.float32


NA = NB // 2


def _fused_kernel(x1_ref, x2_ref, wr_ref, w1_ref, b1_ref, g_ref, beta_ref,
                  w2_ref, b2_ref, out_ref, loss_ref,
                  logits_s, m_s, l_s, s_s, so_s, usage_s):
    i = pl.program_id(0)

    @pl.when(i == 0)
    def _init():
        m_s[...] = jnp.full_like(m_s[...], -1e30)
        l_s[...] = jnp.zeros_like(l_s[...])
        s_s[...] = jnp.zeros_like(s_s[...])
        usage_s[...] = jnp.zeros_like(usage_s[...])

    @pl.when(i < NA)
    def _phase_a():
        wr16 = wr_ref[...].astype(_bf16)
        x1 = x1_ref[...]
        x2 = x2_ref[...]
        logits1 = jax.lax.dot_general(
            wr16, x1.astype(_bf16),
            (((1,), (1,)), ((), ())), preferred_element_type=_f32)
        logits2 = jax.lax.dot_general(
            wr16, x2.astype(_bf16),
            (((1,), (1,)), ((), ())), preferred_element_type=_f32)
        logits_s[2 * i] = logits1
        logits_s[2 * i + 1] = logits2
        m_old = m_s[...]
        bmax = jnp.maximum(jnp.max(logits1, axis=1, keepdims=True),
                           jnp.max(logits2, axis=1, keepdims=True))
        m_new = jnp.maximum(m_old, bmax)
        scale = jnp.exp(m_old - m_new)
        p1 = jnp.exp(logits1 - m_new)
        p2 = jnp.exp(logits2 - m_new)
        m_s[...] = m_new
        l_s[...] = (l_s[...] * scale
                    + jnp.sum(p1, axis=1, keepdims=True)
                    + jnp.sum(p2, axis=1, keepdims=True))
        s_s[...] = (s_s[...] * scale
                    + jax.lax.dot_general(
                        p1.astype(_bf16), x1.astype(_bf16),
                        (((1,), (0,)), ((), ())),
                        preferred_element_type=_f32)
                    + jax.lax.dot_general(
                        p2.astype(_bf16), x2.astype(_bf16),
                        (((1,), (0,)), ((), ())),
                        preferred_element_type=_f32))

        @pl.when(i == NA - 1)
        def _finalize():
            s_s[...] = s_s[...] / l_s[...]

    @pl.when(jnp.logical_and(i >= NA, i < NA + E))
    def _phase_b():
        e = i - NA
        s = s_s[pl.ds(e, 1), :]
        h = jax.lax.dot_general(
            s.astype(_bf16), w1_ref[0].astype(_bf16),
            (((1,), (1,)), ((), ())), preferred_element_type=_f32)
        h = h + b1_ref[0]
        mu = jnp.mean(h, axis=1, keepdims=True)
        var = jnp.mean((h - mu) ** 2, axis=1, keepdims=True)
        h = (h - mu) * jax.lax.rsqrt(var + 1e-5) * g_ref[0] + beta_ref[0]
        h = 0.5 * h * (1.0 + jax.lax.erf(h * (1.0 / math.sqrt(2.0))))
        out = jax.lax.dot_general(
            h.astype(_bf16), w2_ref[0].astype(_bf16),
            (((1,), (1,)), ((), ())), preferred_element_type=_f32)
        so_s[pl.ds(e, 1), :] = out + b2_ref[0]

    @pl.when(i >= NA + E)
    def _phase_c():
        j = i - (NA + E)
        logits = logits_s[j]
        mx = jnp.max(logits, axis=0, keepdims=True)
        p = jnp.exp(logits - mx)
        combine = p / jnp.sum(p, axis=0, keepdims=True)
        out_ref[...] = jax.lax.dot_general(
            combine.astype(_bf16), so_s[...].astype(_bf16),
            (((0,), (0,)), ((), ())), preferred_element_type=_f32)
        usage_s[...] += jnp.sum(combine, axis=1, keepdims=True)

        @pl.when(i == NA + E + NB - 1)
        def _loss():
            rm = jnp.mean(wr_ref[...], axis=1, keepdims=True)
            pe = jnp.exp(rm - jnp.max(rm, axis=0, keepdims=True))
            pe = pe / jnp.sum(pe, axis=0, keepdims=True)
            mean_usage = usage_s[...] / float(T)
            loss_ref[...] = float(E) * jnp.sum(mean_usage * pe, keepdims=True)


@jax.jit
def kernel(x, Wr, W1, b1, g, beta, W2, b2):
    expert_idx = lambda i: (jnp.clip(i - NA, 0, E - 1), 0, 0)
    output, loss = pl.pallas_call(
        _fused_kernel,
        grid=(NA + E + NB,),
        in_specs=[
            pl.BlockSpec((TB, D), lambda i: (jnp.minimum(2 * i, NB - 2), 0)),
            pl.BlockSpec((TB, D),
                         lambda i: (jnp.minimum(2 * i + 1, NB - 1), 0)),
            pl.BlockSpec((E, D), lambda i: (0, 0)),
            pl.BlockSpec((1, H, D), expert_idx),
            pl.BlockSpec((1, 1, H), expert_idx),
            pl.BlockSpec((1, 1, H), expert_idx),
            pl.BlockSpec((1, 1, H), expert_idx),
            pl.BlockSpec((1, D, H), expert_idx),
            pl.BlockSpec((1, 1, D), expert_idx),
        ],
        out_specs=[
            pl.BlockSpec((TB, D), lambda i: (jnp.maximum(i - (NA + E), 0), 0)),
            pl.BlockSpec((1, 1), lambda i: (0, 0)),
        ],
        out_shape=[
            jax.ShapeDtypeStruct((T, D), _f32),
            jax.ShapeDtypeStruct((1, 1), _f32),
        ],
        scratch_shapes=[
            pltpu.VMEM((NB, E, TB), _f32),
            pltpu.VMEM((E, 1), _f32),
            pltpu.VMEM((E, 1), _f32),
            pltpu.VMEM((E, D), _f32),
            pltpu.VMEM((E, D), _f32),
            pltpu.VMEM((E, 1), _f32),
        ],
    )(x, x, Wr, W1, b1.reshape(E, 1, H), g.reshape(E, 1, H),
      beta.reshape(E, 1, H), W2, b2.reshape(E, 1, D))

    return (output, loss.reshape(()))


# speedup vs baseline: 8.0448x; 1.0711x over previous
import math

import jax
import jax.numpy as jnp
from jax.experimental import pallas as pl
from jax.experimental.pallas import tpu as pltpu

E = 8
D = 768
H = 2 * D
T = 32768
TB = 2048
NB = T // TB

_bf16 = jnp.bfloat16
_f32 = jnp.float32


def _fused_kernel(x_ref, wr_ref, w1_ref, b1_ref, g_ref, beta_ref, w2_ref,
                  b2_ref, out_ref, loss_ref,
                  logits_s, m_s, l_s, s_s, so_s, usage_s):
    i = pl.program_id(0)

    @pl.when(i == 0)
    def _init():
        m_s[...] = jnp.full_like(m_s[...], -1e30)
        l_s[...] = jnp.zeros_like(l_s[...])
        s_s[...] = jnp.zeros_like(s_s[...])
        usage_s[...] = jnp.zeros_like(usage_s[...])

    @pl.when(i < NB)
    def _phase_a():
        x = x_ref[...]
        logits = jax.lax.dot_general(
            wr_ref[...].astype(_bf16), x.astype(_bf16),
            (((1,), (1,)), ((), ())), preferred_element_type=_f32)
        logits_s[i] = logits
        m_old = m_s[...]
        bmax = jnp.max(logits, axis=1, keepdims=True)
        m_new = jnp.maximum(m_old, bmax)
        scale = jnp.exp(m_old - m_new)
        p = jnp.exp(logits - m_new)
        m_s[...] = m_new
        l_s[...] = l_s[...] * scale + jnp.sum(p, axis=1, keepdims=True)
        s_s[...] = s_s[...] * scale + jax.lax.dot_general(
            p.astype(_bf16), x.astype(_bf16),
            (((1,), (0,)), ((), ())), preferred_element_type=_f32)

        @pl.when(i == NB - 1)
        def _finalize():
            s_s[...] = s_s[...] / l_s[...]

    @pl.when(jnp.logical_and(i >= NB, i < NB + E))
    def _phase_b():
        e = i - NB
        s = s_s[pl.ds(e, 1), :]
        h = jax.lax.dot_general(
            s.astype(_bf16), w1_ref[0].astype(_bf16),
            (((1,), (1,)), ((), ())), preferred_element_type=_f32)
        h = h + b1_ref[0]
        mu = jnp.mean(h, axis=1, keepdims=True)
        var = jnp.mean((h - mu) ** 2, axis=1, keepdims=True)
        h = (h - mu) * jax.lax.rsqrt(var + 1e-5) * g_ref[0] + beta_ref[0]
        h = 0.5 * h * (1.0 + jax.lax.erf(h * (1.0 / math.sqrt(2.0))))
        out = jax.lax.dot_general(
            h.astype(_bf16), w2_ref[0].astype(_bf16),
            (((1,), (1,)), ((), ())), preferred_element_type=_f32)
        so_s[pl.ds(e, 1), :] = out + b2_ref[0]

    @pl.when(i >= NB + E)
    def _phase_c():
        j = i - (NB + E)
        logits = logits_s[j]
        mx = jnp.max(logits, axis=0, keepdims=True)
        p = jnp.exp(logits - mx)
        combine = p / jnp.sum(p, axis=0, keepdims=True)
        out_ref[...] = jax.lax.dot_general(
            combine.astype(_bf16), so_s[...].astype(_bf16),
            (((0,), (0,)), ((), ())), preferred_element_type=_f32)
        usage_s[...] += jnp.sum(combine, axis=1, keepdims=True)

        @pl.when(i == NB + E + NB - 1)
        def _loss():
            rm = jnp.mean(wr_ref[...], axis=1, keepdims=True)
            pe = jnp.exp(rm - jnp.max(rm, axis=0, keepdims=True))
            pe = pe / jnp.sum(pe, axis=0, keepdims=True)
            mean_usage = usage_s[...] / float(T)
            loss_ref[...] = float(E) * jnp.sum(mean_usage * pe, keepdims=True)


@jax.jit
def kernel(x, Wr, W1, b1, g, beta, W2, b2):
    expert_idx = lambda i: (jnp.clip(i - NB, 0, E - 1), 0, 0)
    output, loss = pl.pallas_call(
        _fused_kernel,
        grid=(NB + E + NB,),
        in_specs=[
            pl.BlockSpec((TB, D), lambda i: (jnp.minimum(i, NB - 1), 0)),
            pl.BlockSpec((E, D), lambda i: (0, 0)),
            pl.BlockSpec((1, H, D), expert_idx),
            pl.BlockSpec((1, 1, H), expert_idx),
            pl.BlockSpec((1, 1, H), expert_idx),
            pl.BlockSpec((1, 1, H), expert_idx),
            pl.BlockSpec((1, D, H), expert_idx),
            pl.BlockSpec((1, 1, D), expert_idx),
        ],
        out_specs=[
            pl.BlockSpec((TB, D), lambda i: (jnp.maximum(i - (NB + E), 0), 0)),
            pl.BlockSpec((1, 1), lambda i: (0, 0)),
        ],
        out_shape=[
            jax.ShapeDtypeStruct((T, D), _f32),
            jax.ShapeDtypeStruct((1, 1), _f32),
        ],
        scratch_shapes=[
            pltpu.VMEM((NB, E, TB), _f32),
            pltpu.VMEM((E, 1), _f32),
            pltpu.VMEM((E, 1), _f32),
            pltpu.VMEM((E, D), _f32),
            pltpu.VMEM((E, D), _f32),
            pltpu.VMEM((E, 1), _f32),
        ],
    )(x, Wr, W1, b1.reshape(E, 1, H), g.reshape(E, 1, H),
      beta.reshape(E, 1, H), W2, b2.reshape(E, 1, D))

    return (output, loss.reshape(()))
